# baseline (speedup 1.0000x reference)
import functools

import jax
import jax.numpy as jnp
from jax import lax
from jax.experimental import pallas as pl
from jax.experimental.pallas import tpu as pltpu
from jax.experimental.pallas import tpu_sc as plsc

B = 16384
D = 128
NC = 2
NS = 16
NW = NC * NS
RPW = B // NW
CH = 64
NCHUNK = RPW // CH
NPAIR = NCHUNK // 2
FB = 8


def _make_sc_kernel():
    mesh = plsc.VectorSubcoreMesh(core_axis_name="c", subcore_axis_name="s")

    @functools.partial(
        pl.kernel,
        mesh=mesh,
        out_type=(
            jax.ShapeDtypeStruct((NW, 16), jnp.float32),
            jax.ShapeDtypeStruct((B,), jnp.float32),
        ),
        compiler_params=pltpu.CompilerParams(needs_layout_passes=False),
        scratch_types=[
            pltpu.VMEM((NCHUNK, CH), jnp.int32),
            pltpu.VMEM((CH, D), jnp.float32),
            pltpu.VMEM((CH, D), jnp.float32),
            pltpu.VMEM((CH, D), jnp.float32),
            pltpu.VMEM((CH, D), jnp.float32),
            pltpu.VMEM((CH,), jnp.float32),
            pltpu.VMEM((CH,), jnp.float32),
            pltpu.VMEM((16,), jnp.float32),
            pltpu.SemaphoreType.DMA,
            pltpu.SemaphoreType.DMA,
            pltpu.SemaphoreType.DMA,
            pltpu.SemaphoreType.DMA,
            pltpu.SemaphoreType.DMA,
            pltpu.SemaphoreType.DMA,
        ],
    )
    def center_dots(feat_h, lab_h, cen_h, out_h, fc_h,
                    idx_all, f0, f1, r0, r1, fcs0, fcs1, acc_v,
                    sg0, sg1, sf0, sf1, sw0, sw1):
        wid = lax.axis_index("s") * NC + lax.axis_index("c")
        base = wid * RPW
        zero16 = jnp.zeros((16,), jnp.float32)
        lane = lax.iota(jnp.int32, 16)
        dists = (1, 2, 4, 8)
        perms = [lane ^ d for d in dists]
        masks = [(lane & d) != 0 for d in dists]

        dnums = lax.GatherDimensionNumbers(
            offset_dims=(), collapsed_slice_dims=(0,), start_index_map=(0,))

        def vperm(x, p):
            return lax.gather(
                x, p[:, None], dimension_numbers=dnums, slice_sizes=(1,),
                mode=lax.GatherScatterMode.PROMISE_IN_BOUNDS)

        def merge(x, y, lv):
            p = perms[lv]
            return jnp.where(masks[lv], y + vperm(y, p), x + vperm(x, p))

        def start(ci, feat_v, rows_v, sg, sf):
            rbase = base + ci * CH
            pltpu.async_copy(cen_h.at[idx_all.at[ci]], rows_v, sg)
            pltpu.async_copy(feat_h.at[pl.ds(rbase, CH)], feat_v, sf)

        def wait_chunk(feat_v, rows_v, sg, sf):
            pltpu.make_async_copy(cen_h.at[idx_all.at[0]], rows_v, sg).wait()
            pltpu.make_async_copy(feat_h.at[pl.ds(base, CH)], feat_v, sf).wait()

        def compute(feat_v, rows_v, fcs_v, cc16):
            def grp(gi, cc):
                row0 = gi * 16
                stack = []
                for p in range(16):
                    rr = row0 + p
                    fc = None
                    for k in range(D // 16):
                        fv = feat_v[rr, pl.ds(k * 16, 16)]
                        cv = rows_v[rr, pl.ds(k * 16, 16)]
                        fc = fv * cv if fc is None else fc + fv * cv
                        cc = cc + cv * cv
                    node = (0, fc)
                    while stack and stack[-1][0] == node[0]:
                        lv, xfc = stack.pop()
                        node = (lv + 1, merge(xfc, node[1], lv))
                    stack.append(node)
                _, fc16 = stack.pop()
                fcs_v[pl.ds(row0, 16)] = fc16
                return cc

            return lax.fori_loop(0, CH // 16, grp, cc16)

        pltpu.sync_copy(lab_h.at[pl.ds(wid * NCHUNK, NCHUNK)], idx_all)
        start(0, f0, r0, sg0, sf0)

        def pair(j, cc):
            start(2 * j + 1, f1, r1, sg1, sf1)
            wait_chunk(f0, r0, sg0, sf0)

            @pl.when(j > 0)
            def _():
                pltpu.make_async_copy(fcs0, fc_h.at[pl.ds(base, CH)], sw0).wait()

            cc = compute(f0, r0, fcs0, cc)
            pltpu.async_copy(fcs0, fc_h.at[pl.ds(base + 2 * j * CH, CH)], sw0)

            @pl.when(j < NPAIR - 1)
            def _():
                start(2 * j + 2, f0, r0, sg0, sf0)

            wait_chunk(f1, r1, sg1, sf1)

            @pl.when(j > 0)
            def _():
                pltpu.make_async_copy(fcs1, fc_h.at[pl.ds(base, CH)], sw1).wait()

            cc = compute(f1, r1, fcs1, cc)
            pltpu.async_copy(
                fcs1, fc_h.at[pl.ds(base + (2 * j + 1) * CH, CH)], sw1)
            return cc

        cc16 = lax.fori_loop(0, NPAIR, pair, zero16)
        pltpu.make_async_copy(fcs0, fc_h.at[pl.ds(base, CH)], sw0).wait()
        pltpu.make_async_copy(fcs1, fc_h.at[pl.ds(base, CH)], sw1).wait()
        acc_v[...] = cc16 * (0.5 / B)
        pltpu.sync_copy(acc_v, out_h.at[wid])

    return center_dots


_center_dots = _make_sc_kernel()

G = B // D


def _ff_body(x_ref, o_ref):
    x = x_ref[...]
    o_ref[...] = jnp.sum(x * x, axis=2)


_row_norms = pl.pallas_call(
    _ff_body,
    out_shape=jax.ShapeDtypeStruct((G, D), jnp.float32),
    grid=(G // FB,),
    in_specs=[pl.BlockSpec((FB, D, D), lambda i: (i, 0, 0))],
    out_specs=pl.BlockSpec((FB, D), lambda i: (i, 0)),
)


def _combine_body(ff_ref, fc_ref, part_ref, o_ref):
    ff = ff_ref[...]
    fc = fc_ref[...]
    inv = jnp.minimum(lax.rsqrt(jnp.maximum(ff, 1e-24)), 1e12)
    loss = jnp.sum(ff * inv * inv - 2.0 * fc * inv) * (0.5 / B)
    o_ref[...] = jnp.reshape(loss + jnp.sum(part_ref[...]), (1, 1))


_combine = pl.pallas_call(
    _combine_body,
    out_shape=jax.ShapeDtypeStruct((1, 1), jnp.float32),
)


@jax.jit
def kernel(features, labels, centers):
    lab2 = labels.astype(jnp.int32).reshape(B // CH, CH)
    ff = _row_norms(features.reshape(G, D, D))
    partials, fcrow = _center_dots(features, lab2, centers)
    return _combine(ff, fcrow.reshape(G, D), partials)[0, 0]


# speedup vs baseline: 1.1603x; 1.1603x over previous
import functools

import jax
import jax.numpy as jnp
from jax import lax
from jax.experimental import pallas as pl
from jax.experimental.pallas import tpu as pltpu
from jax.experimental.pallas import tpu_sc as plsc

B = 16384
D = 128
NC = 2
NS = 16
NW = NC * NS
RPW = B // NW
CH = 64
NCHUNK = RPW // CH


def _rsqrt_newton(x):
    i = plsc.bitcast(x, jnp.int32)
    i = jnp.int32(0x5F3759DF) - (i >> 1)
    y = plsc.bitcast(i, jnp.float32)
    for _ in range(3):
        y = y * (1.5 - 0.5 * x * y * y)
    return y


def _make_kernel():
    mesh = plsc.VectorSubcoreMesh(core_axis_name="c", subcore_axis_name="s")

    @functools.partial(
        pl.kernel,
        mesh=mesh,
        out_type=jax.ShapeDtypeStruct((NW, 16), jnp.float32),
        compiler_params=pltpu.CompilerParams(needs_layout_passes=False),
        scratch_types=[
            pltpu.VMEM((NCHUNK, CH), jnp.int32),
            pltpu.VMEM((CH, D), jnp.float32),
            pltpu.VMEM((CH, D), jnp.float32),
            pltpu.VMEM((CH, D), jnp.float32),
            pltpu.VMEM((CH, D), jnp.float32),
            pltpu.VMEM((16,), jnp.float32),
            pltpu.SemaphoreType.DMA,
            pltpu.SemaphoreType.DMA,
            pltpu.SemaphoreType.DMA,
            pltpu.SemaphoreType.DMA,
        ],
    )
    def center_loss(feat_h, lab_h, cen_h, out_h,
                    idx_all, f0, f1, r0, r1, acc_v,
                    sg0, sg1, sf0, sf1):
        wid = lax.axis_index("s") * NC + lax.axis_index("c")
        base = wid * RPW
        zero16 = jnp.zeros((16,), jnp.float32)
        lane = lax.iota(jnp.int32, 16)
        dists = (1, 2, 4, 8)
        perms = [lane ^ d for d in dists]
        masks = [(lane & d) != 0 for d in dists]

        dnums = lax.GatherDimensionNumbers(
            offset_dims=(), collapsed_slice_dims=(0,), start_index_map=(0,))

        def vperm(x, p):
            return lax.gather(
                x, p[:, None], dimension_numbers=dnums, slice_sizes=(1,),
                mode=lax.GatherScatterMode.PROMISE_IN_BOUNDS)

        def merge(x, y, lv):
            p = perms[lv]
            return jnp.where(masks[lv], y + vperm(y, p), x + vperm(x, p))

        def start(ci, feat_v, rows_v, sg, sf):
            rbase = base + ci * CH
            pltpu.async_copy(cen_h.at[idx_all.at[ci]], rows_v, sg)
            pltpu.async_copy(feat_h.at[pl.ds(rbase, CH)], feat_v, sf)

        def wait_chunk(feat_v, rows_v, sg, sf):
            pltpu.make_async_copy(cen_h.at[idx_all.at[0]], rows_v, sg).wait()
            pltpu.make_async_copy(feat_h.at[pl.ds(base, CH)], feat_v, sf).wait()

        def compute(feat_v, rows_v, loss16, cc16):
            def grp(gi, carry):
                l16, cc = carry
                row0 = gi * 16
                ff16 = zero16
                fc16 = zero16
                for p in range(16):
                    rr = row0 + p
                    ff = None
                    fc = None
                    for k in range(D // 16):
                        fv = feat_v[rr, pl.ds(k * 16, 16)]
                        cv = rows_v[rr, pl.ds(k * 16, 16)]
                        ff = fv * fv if ff is None else ff + fv * fv
                        fc = fv * cv if fc is None else fc + fv * cv
                        cc = cc + cv * cv
                    sel = lane == p
                    ff16 = jnp.where(sel, jnp.sum(ff), ff16)
                    fc16 = jnp.where(sel, jnp.sum(fc), fc16)
                inv = jnp.minimum(_rsqrt_newton(ff16), 1e12)
                return l16 + ff16 * inv * inv - 2.0 * fc16 * inv, cc

            return lax.fori_loop(0, CH // 16, grp, (loss16, cc16))

        pltpu.sync_copy(lab_h.at[pl.ds(wid * NCHUNK, NCHUNK)], idx_all)
        start(0, f0, r0, sg0, sf0)

        def pair(j, carry):
            l16, cc = carry
            start(2 * j + 1, f1, r1, sg1, sf1)
            wait_chunk(f0, r0, sg0, sf0)
            l16, cc = compute(f0, r0, l16, cc)

            @pl.when(j < (NCHUNK // 2) - 1)
            def _():
                start(2 * j + 2, f0, r0, sg0, sf0)

            wait_chunk(f1, r1, sg1, sf1)
            l16, cc = compute(f1, r1, l16, cc)
            return l16, cc

        loss16, cc16 = lax.fori_loop(0, NCHUNK // 2, pair, (zero16, zero16))
        acc_v[...] = (loss16 + cc16) * (0.5 / B)
        pltpu.sync_copy(acc_v, out_h.at[wid])

    return center_loss


_center_loss = _make_kernel()


@jax.jit
def kernel(features, labels, centers):
    lab2 = labels.astype(jnp.int32).reshape(B // CH, CH)
    partials = _center_loss(features, lab2, centers)
    return jnp.sum(partials)
